# Initial kernel scaffold; baseline (speedup 1.0000x reference)
import jax
import jax.numpy as jnp
from jax.experimental import pallas as pl


def kernel(edge_features, node_features, atom_type, kpoints, edge_index, edge_cell_shift):
    raise NotImplementedError("write your pallas kernel here")


# speedup vs baseline: 10.5969x; 10.5969x over previous
import dataclasses

import jax
import jax.numpy as jnp
from jax import lax
from jax.experimental import pallas as pl
from jax.experimental.pallas import tpu as pltpu
from jax.experimental.pallas import tpu_sc as plsc

N = 256
E = 8192
K = 8
NORB = 4
ROWS = N * NORB
NPS = 16
FCOLS = NPS * NORB * 2
ARH = ROWS // 2
CH = 128


def _tc_prep_body(ef_ref, nf_ref, kp_ref, rs_ref, hp_o, ons_o):
    ef = ef_ref[...]
    z = jnp.zeros((E, 1), jnp.float32)
    theta = 2.0 * jnp.pi * lax.dot_general(
        rs_ref[...], kp_ref[...], (((1,), (1,)), ((), ())),
        preferred_element_type=jnp.float32)
    hp_o[...] = jnp.concatenate(
        [0.5 * ef[:, 0:1], ef[:, 1:4],
         z, 0.5 * ef[:, 4:7],
         z, 0.5 * ef[:, 7:10],
         z, 0.5 * ef[:, 10:13],
         jnp.cos(theta), -jnp.sin(theta)], axis=1)

    nf = nf_ref[...]

    def c(i):
        return nf[:, i:i + 1]

    ons_o[...] = jnp.concatenate(
        [c(0), c(1), c(2), c(3),
         c(1), c(4), 0.5 * (c(5) + c(7)), 0.5 * (c(6) + c(10)),
         c(2), 0.5 * (c(7) + c(5)), c(8), 0.5 * (c(9) + c(11)),
         c(3), 0.5 * (c(10) + c(6)), 0.5 * (c(11) + c(9)), c(12)], axis=1)


def _tc_prep(edge_features, node_features, kpoints, edge_cell_shift):
    return pl.pallas_call(
        _tc_prep_body,
        out_shape=[
            jax.ShapeDtypeStruct((E, 32), jnp.float32),
            jax.ShapeDtypeStruct((N, 16), jnp.float32),
        ],
    )(edge_features, node_features, kpoints, edge_cell_shift)


def _sc_body(hp_hbm, ons_hbm, src_hbm, dst_hbm, out_hbm,
             acc, src_v, dst_v, bd_v, bh_v, hpc, cidx, onsv):
    rh = lax.axis_index("c")
    cs = lax.axis_index("s")
    lo = cs * NPS
    orow0 = rh * ARH

    lane = jnp.arange(16, dtype=jnp.int32)
    af = lane >> 2
    bf = lane & 3
    af2 = af * 2
    bf2 = bf * 2
    zf = jnp.zeros((16,), jnp.float32)
    zi = jnp.zeros((16,), jnp.int32)

    pltpu.sync_copy(src_hbm, src_v)
    pltpu.sync_copy(dst_hbm, dst_v)
    pltpu.sync_copy(ons_hbm.at[pl.ds(lo, NPS)], onsv)

    def zlists(g, carry):
        bd_v[pl.ds(g * 16, 16)] = zi
        bh_v[pl.ds(g * 16, 16)] = zi
        return carry

    lax.fori_loop(0, (E + 16) // 16, zlists, 0)

    def zcidx(g, carry):
        cidx[pl.ds(g * 16, 16)] = zi
        return carry

    lax.fori_loop(0, CH // 16, zcidx, 0)

    def build(g, carry):
        pd, ph_ = carry
        s16 = src_v[pl.ds(g * 16, 16)]
        d16 = dst_v[pl.ds(g * 16, 16)]
        e16 = g * 16 + lane
        epack = ((e16 >> 2) << 19) | ((e16 & 3) << 17)
        dloc = d16 - lo
        sloc = s16 - lo
        srow = s16 * 4 - orow0
        drow = d16 * 4 - orow0
        md = (dloc >= 0) & (dloc < NPS) & (srow >= 0) & (srow < ARH)
        mh = (sloc >= 0) & (sloc < NPS) & (drow >= 0) & (drow < ARH)
        mdi = md.astype(jnp.int32)
        mhi = mh.astype(jnp.int32)
        posd = pd + jnp.cumsum(mdi) - 1
        posh = ph_ + jnp.cumsum(mhi) - 1
        plsc.store_scatter(bd_v, [posd],
                           epack | (srow << 7) | (dloc * 8), mask=md)
        plsc.store_scatter(bh_v, [posh],
                           epack | (drow << 7) | (sloc * 8), mask=mh)
        return pd + jnp.sum(mdi), ph_ + jnp.sum(mhi)

    nd, nh = lax.fori_loop(0, E // 16, build,
                           (jnp.int32(0), jnp.int32(0)))

    dummy = jnp.full((16,), ARH << 7, jnp.int32)
    plsc.store_scatter(bd_v, [nd + lane], dummy)
    plsc.store_scatter(bh_v, [nh + lane], dummy)
    ndp = ((nd + 15) // 16) * 16
    nhp = ((nh + 15) // 16) * 16

    def process(kk, base_v, n_edges, hermitian):
        def chunk(j, carry):
            c0 = j * CH

            def mkidx(g, cc):
                cidx[pl.ds(g * 16, 16)] = base_v[pl.ds(c0 + g * 16, 16)] >> 19
                return cc

            lax.fori_loop(0, CH // 16, mkidx, 0)
            pltpu.sync_copy(hp_hbm.at[cidx], hpc)
            cnt = jnp.minimum(CH, n_edges - c0)

            def group(g, gc):
                bv = base_v[pl.ds(c0 + g * 16, 16)]
                for u in range(16):
                    b = bv[u]
                    rb = (b >> 7) & 1023
                    cb = b & 127
                    sub32 = (b >> 12) & 96
                    i = g * 16 + u
                    hv = hpc[i, pl.ds(sub32, 16)]
                    phv = hpc[i, pl.ds(sub32 + 16, 16)]
                    vre = hv * phv[kk]
                    vim = hv * phv[kk + 8]
                    if hermitian:
                        rowv = rb + bf
                        colv = cb + af2
                        vim = -vim
                    else:
                        rowv = rb + af
                        colv = cb + bf2
                    plsc.addupdate_scatter(acc, [rowv, colv], vre)
                    plsc.addupdate_scatter(acc, [rowv, colv + 1], vim)
                return gc

            lax.fori_loop(0, cnt // 16, group, 0)
            return carry

        lax.fori_loop(0, (n_edges + CH - 1) // CH, chunk, 0)

    for kk in range(K):
        def zacc(r2, carry):
            for rr in range(2):
                for cc in range(8):
                    acc[r2 * 2 + rr, pl.ds(cc * 16, 16)] = zf
            return carry

        lax.fori_loop(0, ARH // 2, zacc, 0)

        for il in range(NPS):
            node = lo + il
            rbase = jnp.where((node * 4 >= orow0) & (node * 4 < orow0 + ARH),
                              node * 4 - orow0, ARH)
            rowv = rbase + af
            colv = il * 8 + bf2
            plsc.addupdate_scatter(acc, [rowv, colv], onsv[il, :])

        process(kk, bd_v, ndp, hermitian=False)
        process(kk, bh_v, nhp, hermitian=True)

        pltpu.sync_copy(acc.at[pl.ds(0, ARH)],
                        out_hbm.at[kk, pl.ds(orow0, ARH),
                                   pl.ds(cs * FCOLS, FCOLS)])


def _sc_scatter(hp, ons16, src, dst):
    mesh = plsc.VectorSubcoreMesh(core_axis_name="c", subcore_axis_name="s")
    cp = pltpu.CompilerParams()
    if "needs_layout_passes" in pltpu.CompilerParams.__dataclass_fields__:
        cp = dataclasses.replace(cp, needs_layout_passes=False)
    kern = pl.kernel(
        _sc_body,
        out_type=jax.ShapeDtypeStruct((K, ROWS, 2 * ROWS), jnp.float32),
        mesh=mesh,
        compiler_params=cp,
        scratch_types=[
            pltpu.VMEM((ARH + 8, FCOLS), jnp.float32),
            pltpu.VMEM((E,), jnp.int32),
            pltpu.VMEM((E,), jnp.int32),
            pltpu.VMEM((E + 16,), jnp.int32),
            pltpu.VMEM((E + 16,), jnp.int32),
            pltpu.VMEM((CH, 128), jnp.float32),
            pltpu.VMEM((CH,), jnp.int32),
            pltpu.VMEM((NPS, 16), jnp.float32),
        ],
    )
    return kern(hp, ons16, src, dst)


def kernel(edge_features, node_features, atom_type, kpoints, edge_index,
           edge_cell_shift):
    del atom_type
    hp, ons16 = _tc_prep(
        edge_features.astype(jnp.float32),
        node_features.astype(jnp.float32),
        kpoints.astype(jnp.float32),
        edge_cell_shift.astype(jnp.float32))
    hp = hp.reshape(E // 4, 128)
    src = edge_index[0].astype(jnp.int32)
    dst = edge_index[1].astype(jnp.int32)
    outf = _sc_scatter(hp, ons16, src, dst)
    outf = outf.reshape(K, ROWS, ROWS, 2)
    return lax.complex(outf[..., 0], outf[..., 1])
